# baseline (speedup 1.0000x reference)
import functools

import jax
import jax.numpy as jnp
from jax import lax
from jax.experimental import pallas as pl
from jax.experimental.pallas import tpu as pltpu
from jax.experimental.pallas import tpu_sc as plsc

N = 10000
D = 128
E = 320000
NC, NS = 2, 16
NW = NC * NS
NP = 10240
CH = 128
NCHUNK = E // CH
NJ = NCHUNK // NW
NEXTRA = NCHUNK - NJ * NW
G = 6
NGRP = NJ // G
PERS = NP // NS

_mesh = plsc.VectorSubcoreMesh(core_axis_name="c", subcore_axis_name="s")


@functools.partial(
    pl.kernel,
    out_type=jax.ShapeDtypeStruct((NC, NP), jnp.float32),
    mesh=_mesh,
    scratch_types=[
        pltpu.VMEM((NJ + 1, CH), jnp.int32),
        pltpu.VMEM((CH,), jnp.float32),
        pltpu.VMEM_SHARED((NP,), jnp.float32),
        pltpu.SemaphoreType.DMA,
    ],
)
def _deg_kernel(ef_hbm, ones_hbm, zeros_hbm, out_hbm, idx_v, ones_v, deg_sh,
                sem):
    c = lax.axis_index("c")
    s = lax.axis_index("s")
    w = c * NS + s
    nj = jnp.where(w < NEXTRA, NJ + 1, NJ)

    pltpu.async_copy(ones_hbm, ones_v, sem)
    pltpu.sync_copy(zeros_hbm, deg_sh.at[pl.ds(s * PERS, PERS)])
    pltpu.make_async_copy(ones_hbm, ones_v, sem).wait()

    def load(r, _):
        pltpu.async_copy(ef_hbm.at[1, pl.ds(pl.multiple_of((w * NJ + r) * CH, CH), CH)],
                         idx_v.at[r], sem)
        return 0
    lax.fori_loop(0, NJ, load, 0)

    @pl.when(w < NEXTRA)
    def _():
        pltpu.async_copy(ef_hbm.at[1, pl.ds(pl.multiple_of((NW * NJ + w) * CH, CH), CH)],
                         idx_v.at[NJ], sem)

    def drain(r, _):
        pltpu.make_async_copy(ef_hbm.at[1, pl.ds(0, CH)], idx_v.at[0],
                              sem).wait()
        return 0
    lax.fori_loop(0, nj, drain, 0)
    plsc.subcore_barrier()

    def body(j, _):
        pltpu.sync_copy(ones_v, deg_sh.at[idx_v.at[j]], add=True)
        return 0
    lax.fori_loop(0, nj, body, 0)

    plsc.subcore_barrier()
    pltpu.sync_copy(deg_sh.at[pl.ds(s * PERS, PERS)],
                    out_hbm.at[c, pl.ds(s * PERS, PERS)])


@functools.partial(
    pl.kernel,
    out_type=jax.ShapeDtypeStruct((NC, NP, D), jnp.float32),
    mesh=_mesh,
    scratch_types=[
        pltpu.VMEM((2, G, CH), jnp.int32),
        pltpu.VMEM((2, G, CH), jnp.int32),
        pltpu.VMEM((2, CH, D), jnp.float32),
        pltpu.VMEM_SHARED((NP, D), jnp.float32),
        pltpu.SemaphoreType.DMA,
        pltpu.SemaphoreType.DMA,
        pltpu.SemaphoreType.DMA,
        pltpu.SemaphoreType.DMA,
        pltpu.SemaphoreType.DMA,
    ],
)
def _agg_kernel(y_hbm, ef_hbm, zeros_hbm, out_hbm,
                si_v, di_v, rows_v, acc_sh, semi, sem0, sem1, ssem0, ssem1):
    c = lax.axis_index("c")
    s = lax.axis_index("s")
    w = c * NS + s
    sems = (sem0, sem1)
    ssems = (ssem0, ssem1)

    def fetch_group(g, slot):
        for r in range(G):
            pltpu.async_copy(
                ef_hbm.at[0, pl.ds(pl.multiple_of((w * NJ + g * G + r) * CH, CH), CH)],
                si_v.at[slot, r], semi)
            pltpu.async_copy(
                ef_hbm.at[1, pl.ds(pl.multiple_of((w * NJ + g * G + r) * CH, CH), CH)],
                di_v.at[slot, r], semi)

    def wait_group():
        for _ in range(2 * G):
            pltpu.make_async_copy(ef_hbm.at[0, pl.ds(0, CH)], si_v.at[0, 0],
                                  semi).wait()

    fetch_group(0, 0)
    pltpu.sync_copy(zeros_hbm, acc_sh.at[pl.ds(s * PERS, PERS)])
    wait_group()
    plsc.subcore_barrier()
    pltpu.async_copy(y_hbm.at[si_v.at[0, 0]], rows_v.at[0], sem0)

    def group(g, _):
        gb = lax.rem(g, 2)
        gn = lax.rem(g + 1, 2)

        @pl.when(g + 1 < NGRP)
        def _():
            fetch_group(g + 1, gn)

        for j in range(G):
            b = j % 2
            nb = (j + 1) % 2
            pltpu.make_async_copy(y_hbm.at[si_v.at[gb, j]], rows_v.at[b],
                                  sems[b]).wait()
            pltpu.async_copy(rows_v.at[b], acc_sh.at[di_v.at[gb, j]],
                             ssems[b], add=True)
            if j == 0:
                @pl.when(g > 0)
                def _():
                    pltpu.make_async_copy(rows_v.at[nb],
                                          acc_sh.at[di_v.at[0, 0]],
                                          ssems[nb]).wait()
            else:
                pltpu.make_async_copy(rows_v.at[nb],
                                      acc_sh.at[di_v.at[0, 0]],
                                      ssems[nb]).wait()
            if j + 1 < G:
                pltpu.async_copy(y_hbm.at[si_v.at[gb, j + 1]], rows_v.at[nb],
                                 sems[nb])
            else:
                @pl.when(g + 1 < NGRP)
                def _():
                    wait_group()
                    pltpu.async_copy(y_hbm.at[si_v.at[gn, 0]], rows_v.at[nb],
                                     sems[nb])
        return 0
    lax.fori_loop(0, NGRP, group, 0)

    pltpu.make_async_copy(rows_v.at[1], acc_sh.at[di_v.at[0, 0]],
                          ssems[1]).wait()

    @pl.when(w < NEXTRA)
    def _():
        pltpu.async_copy(ef_hbm.at[0, pl.ds(pl.multiple_of((NW * NJ + w) * CH, CH), CH)],
                         si_v.at[0, 0], semi)
        pltpu.async_copy(ef_hbm.at[1, pl.ds(pl.multiple_of((NW * NJ + w) * CH, CH), CH)],
                         di_v.at[0, 0], semi)
        pltpu.make_async_copy(ef_hbm.at[0, pl.ds(0, CH)], si_v.at[0, 0],
                              semi).wait()
        pltpu.make_async_copy(ef_hbm.at[0, pl.ds(0, CH)], si_v.at[0, 0],
                              semi).wait()
        pltpu.async_copy(y_hbm.at[si_v.at[0, 0]], rows_v.at[0], sem0).wait()
        pltpu.sync_copy(rows_v.at[0], acc_sh.at[di_v.at[0, 0]], add=True)

    plsc.subcore_barrier()
    pltpu.sync_copy(acc_sh.at[pl.ds(s * PERS, PERS)],
                    out_hbm.at[c, pl.ds(s * PERS, PERS)])


BLK = 1000
H = 512
C = 79


def _xw_body(x_ref, w_ref, o_ref):
    o_ref[...] = jnp.dot(x_ref[...], w_ref[...],
                         preferred_element_type=jnp.float32)


def _xw_call(x, W_gcn):
    return pl.pallas_call(
        _xw_body,
        grid=(N // BLK,),
        in_specs=[
            pl.BlockSpec((BLK, D), lambda i: (i, 0)),
            pl.BlockSpec((D, D), lambda i: (0, 0)),
        ],
        out_specs=pl.BlockSpec((BLK, D), lambda i: (i, 0)),
        out_shape=jax.ShapeDtypeStruct((N, D), jnp.float32),
    )(x, W_gcn)


def _scale_body(xw_ref, degT_ref, y_ref):
    deg = degT_ref[...]
    dis = lax.rsqrt(deg[:, 0:1] + deg[:, 1:2] + 1.0)
    y_ref[...] = xw_ref[...] * dis


def _scale_call(xw, degT):
    return pl.pallas_call(
        _scale_body,
        grid=(N // BLK,),
        in_specs=[
            pl.BlockSpec((BLK, D), lambda i: (i, 0)),
            pl.BlockSpec((BLK, 2), lambda i: (i, 0)),
        ],
        out_specs=pl.BlockSpec((BLK, D), lambda i: (i, 0)),
        out_shape=jax.ShapeDtypeStruct((N, D), jnp.float32),
    )(xw, degT)


def _mlp_body(x_ref, y_ref, acc_ref, degT_ref, bg_ref, wh_ref, bh_ref,
              wc_ref, bc_ref, h_ref, l_ref):
    deg = degT_ref[...]
    dis = lax.rsqrt(deg[:, 0:1] + deg[:, 1:2] + 1.0)
    agg = (acc_ref[0] + acc_ref[1] + y_ref[...]) * dis + bg_ref[...]
    x1 = jnp.maximum(agg, 0.0)
    cat = jnp.concatenate([x_ref[...], x1], axis=1)
    h = jnp.dot(cat, wh_ref[...], preferred_element_type=jnp.float32)
    h = jnp.maximum(h + bh_ref[...], 0.0)
    h_ref[...] = h
    l_ref[...] = jnp.dot(h, wc_ref[...],
                         preferred_element_type=jnp.float32) + bc_ref[...]


def _mlp_call(x, y, accp, degT, b_gcn, W_h, b_h, W_c, b_c):
    return pl.pallas_call(
        _mlp_body,
        grid=(N // BLK,),
        in_specs=[
            pl.BlockSpec((BLK, D), lambda i: (i, 0)),
            pl.BlockSpec((BLK, D), lambda i: (i, 0)),
            pl.BlockSpec((NC, BLK, D), lambda i: (0, i, 0)),
            pl.BlockSpec((BLK, 2), lambda i: (i, 0)),
            pl.BlockSpec((D,), lambda i: (0,)),
            pl.BlockSpec((2 * D, H), lambda i: (0, 0)),
            pl.BlockSpec((H,), lambda i: (0,)),
            pl.BlockSpec((H, C), lambda i: (0, 0)),
            pl.BlockSpec((C,), lambda i: (0,)),
        ],
        out_specs=[
            pl.BlockSpec((BLK, H), lambda i: (i, 0)),
            pl.BlockSpec((BLK, C), lambda i: (i, 0)),
        ],
        out_shape=[
            jax.ShapeDtypeStruct((N, H), jnp.float32),
            jax.ShapeDtypeStruct((N, C), jnp.float32),
        ],
    )(x, y, accp, degT, b_gcn, W_h, b_h, W_c, b_c)


def kernel(x, edge_index, img_sizes, W_gcn, b_gcn, W_h, b_h, W_c, b_c):
    ei = edge_index.astype(jnp.int32)
    ones_c = jnp.ones((CH,), jnp.float32)
    zeros_c = jnp.zeros((PERS,), jnp.float32)
    zrows_c = jnp.zeros((PERS, D), jnp.float32)
    xw = _xw_call(x, W_gcn)
    degp = _deg_kernel(ei, ones_c, zeros_c)
    degT = jnp.swapaxes(degp, 0, 1)
    y = _scale_call(xw, degT)
    accp = _agg_kernel(y, ei, zrows_c)
    h, logits = _mlp_call(x, y, accp, degT,
                          b_gcn, W_h, b_h, W_c, b_c)
    return (h, logits)


# speedup vs baseline: 1.0071x; 1.0071x over previous
import functools

import jax
import jax.numpy as jnp
from jax import lax
from jax.experimental import pallas as pl
from jax.experimental.pallas import tpu as pltpu
from jax.experimental.pallas import tpu_sc as plsc

N = 10000
D = 128
E = 320000
NC, NS = 2, 16
NW = NC * NS
NP = 10240
CH = 128
NCHUNK = E // CH
NJ = NCHUNK // NW
NEXTRA = NCHUNK - NJ * NW
G = 6
NGRP = NJ // G
PERS = NP // NS

_mesh = plsc.VectorSubcoreMesh(core_axis_name="c", subcore_axis_name="s")


@functools.partial(
    pl.kernel,
    out_type=jax.ShapeDtypeStruct((NC, NP), jnp.float32),
    mesh=_mesh,
    scratch_types=[
        pltpu.VMEM((NJ + 1, CH), jnp.int32),
        pltpu.VMEM((CH,), jnp.float32),
        pltpu.VMEM_SHARED((NP,), jnp.float32),
        pltpu.SemaphoreType.DMA,
    ],
)
def _deg_kernel(ef_hbm, ones_hbm, zeros_hbm, out_hbm, idx_v, ones_v, deg_sh,
                sem):
    c = lax.axis_index("c")
    s = lax.axis_index("s")
    w = c * NS + s
    nj = jnp.where(w < NEXTRA, NJ + 1, NJ)

    pltpu.async_copy(ones_hbm, ones_v, sem)
    pltpu.sync_copy(zeros_hbm, deg_sh.at[pl.ds(s * PERS, PERS)])
    pltpu.make_async_copy(ones_hbm, ones_v, sem).wait()

    def load(r, _):
        pltpu.async_copy(ef_hbm.at[1, pl.ds(pl.multiple_of((w * NJ + r) * CH, CH), CH)],
                         idx_v.at[r], sem)
        return 0
    lax.fori_loop(0, NJ, load, 0)

    @pl.when(w < NEXTRA)
    def _():
        pltpu.async_copy(ef_hbm.at[1, pl.ds(pl.multiple_of((NW * NJ + w) * CH, CH), CH)],
                         idx_v.at[NJ], sem)

    def drain(r, _):
        pltpu.make_async_copy(ef_hbm.at[1, pl.ds(0, CH)], idx_v.at[0],
                              sem).wait()
        return 0
    lax.fori_loop(0, nj, drain, 0)
    plsc.subcore_barrier()

    def body(j, _):
        pltpu.sync_copy(ones_v, deg_sh.at[idx_v.at[j]], add=True)
        return 0
    lax.fori_loop(0, nj, body, 0)

    plsc.subcore_barrier()
    pltpu.sync_copy(deg_sh.at[pl.ds(s * PERS, PERS)],
                    out_hbm.at[c, pl.ds(s * PERS, PERS)])


@functools.partial(
    pl.kernel,
    out_type=jax.ShapeDtypeStruct((NC, NP, D), jnp.float32),
    mesh=_mesh,
    scratch_types=[
        pltpu.VMEM((2, G, CH), jnp.int32),
        pltpu.VMEM((2, G, CH), jnp.int32),
        pltpu.VMEM((2, CH, D), jnp.float32),
        pltpu.VMEM_SHARED((NP, D), jnp.float32),
        pltpu.SemaphoreType.DMA,
        pltpu.SemaphoreType.DMA,
        pltpu.SemaphoreType.DMA,
        pltpu.SemaphoreType.DMA,
        pltpu.SemaphoreType.DMA,
    ],
)
def _agg_kernel(y_hbm, ef_hbm, zeros_hbm, out_hbm,
                si_v, di_v, rows_v, acc_sh, semi, sem0, sem1, ssem0, ssem1):
    c = lax.axis_index("c")
    s = lax.axis_index("s")
    w = c * NS + s
    sems = (sem0, sem1)
    ssems = (ssem0, ssem1)

    def fetch_group(g, slot):
        for r in range(G):
            pltpu.async_copy(
                ef_hbm.at[0, pl.ds(pl.multiple_of((w * NJ + g * G + r) * CH, CH), CH)],
                si_v.at[slot, r], semi)
            pltpu.async_copy(
                ef_hbm.at[1, pl.ds(pl.multiple_of((w * NJ + g * G + r) * CH, CH), CH)],
                di_v.at[slot, r], semi)

    def wait_group():
        for _ in range(2 * G):
            pltpu.make_async_copy(ef_hbm.at[0, pl.ds(0, CH)], si_v.at[0, 0],
                                  semi).wait()

    fetch_group(0, 0)
    pltpu.async_copy(zeros_hbm, rows_v.at[0], sem0)
    pltpu.make_async_copy(zeros_hbm, rows_v.at[0], sem0).wait()

    def zcopy(k, _):
        pltpu.sync_copy(rows_v.at[0], acc_sh.at[pl.ds(s * PERS + k * CH, CH)])
        return 0
    lax.fori_loop(0, PERS // CH, zcopy, 0)
    wait_group()
    plsc.subcore_barrier()
    pltpu.async_copy(y_hbm.at[si_v.at[0, 0]], rows_v.at[0], sem0)

    def group(g, _):
        gb = lax.rem(g, 2)
        gn = lax.rem(g + 1, 2)

        @pl.when(g + 1 < NGRP)
        def _():
            fetch_group(g + 1, gn)

        for j in range(G):
            b = j % 2
            nb = (j + 1) % 2
            pltpu.make_async_copy(y_hbm.at[si_v.at[gb, j]], rows_v.at[b],
                                  sems[b]).wait()
            pltpu.async_copy(rows_v.at[b], acc_sh.at[di_v.at[gb, j]],
                             ssems[b], add=True)
            if j == 0:
                @pl.when(g > 0)
                def _():
                    pltpu.make_async_copy(rows_v.at[nb],
                                          acc_sh.at[di_v.at[0, 0]],
                                          ssems[nb]).wait()
            else:
                pltpu.make_async_copy(rows_v.at[nb],
                                      acc_sh.at[di_v.at[0, 0]],
                                      ssems[nb]).wait()
            if j + 1 < G:
                pltpu.async_copy(y_hbm.at[si_v.at[gb, j + 1]], rows_v.at[nb],
                                 sems[nb])
            else:
                @pl.when(g + 1 < NGRP)
                def _():
                    wait_group()
                    pltpu.async_copy(y_hbm.at[si_v.at[gn, 0]], rows_v.at[nb],
                                     sems[nb])
        return 0
    lax.fori_loop(0, NGRP, group, 0)

    pltpu.make_async_copy(rows_v.at[1], acc_sh.at[di_v.at[0, 0]],
                          ssems[1]).wait()

    @pl.when(w < NEXTRA)
    def _():
        pltpu.async_copy(ef_hbm.at[0, pl.ds(pl.multiple_of((NW * NJ + w) * CH, CH), CH)],
                         si_v.at[0, 0], semi)
        pltpu.async_copy(ef_hbm.at[1, pl.ds(pl.multiple_of((NW * NJ + w) * CH, CH), CH)],
                         di_v.at[0, 0], semi)
        pltpu.make_async_copy(ef_hbm.at[0, pl.ds(0, CH)], si_v.at[0, 0],
                              semi).wait()
        pltpu.make_async_copy(ef_hbm.at[0, pl.ds(0, CH)], si_v.at[0, 0],
                              semi).wait()
        pltpu.async_copy(y_hbm.at[si_v.at[0, 0]], rows_v.at[0], sem0).wait()
        pltpu.sync_copy(rows_v.at[0], acc_sh.at[di_v.at[0, 0]], add=True)

    plsc.subcore_barrier()
    pltpu.sync_copy(acc_sh.at[pl.ds(s * PERS, PERS)],
                    out_hbm.at[c, pl.ds(s * PERS, PERS)])


BLK = 1000
H = 512
C = 79


def _xw_body(x_ref, w_ref, o_ref):
    o_ref[...] = jnp.dot(x_ref[...], w_ref[...],
                         preferred_element_type=jnp.float32)


def _xw_call(x, W_gcn):
    return pl.pallas_call(
        _xw_body,
        grid=(N // BLK,),
        in_specs=[
            pl.BlockSpec((BLK, D), lambda i: (i, 0)),
            pl.BlockSpec((D, D), lambda i: (0, 0)),
        ],
        out_specs=pl.BlockSpec((BLK, D), lambda i: (i, 0)),
        out_shape=jax.ShapeDtypeStruct((N, D), jnp.float32),
    )(x, W_gcn)


def _scale_body(xw_ref, degT_ref, y_ref):
    deg = degT_ref[...]
    dis = lax.rsqrt(deg[:, 0:1] + deg[:, 1:2] + 1.0)
    y_ref[...] = xw_ref[...] * dis


def _scale_call(xw, degT):
    return pl.pallas_call(
        _scale_body,
        grid=(N // BLK,),
        in_specs=[
            pl.BlockSpec((BLK, D), lambda i: (i, 0)),
            pl.BlockSpec((BLK, 2), lambda i: (i, 0)),
        ],
        out_specs=pl.BlockSpec((BLK, D), lambda i: (i, 0)),
        out_shape=jax.ShapeDtypeStruct((N, D), jnp.float32),
    )(xw, degT)


def _mlp_body(x_ref, y_ref, acc_ref, degT_ref, bg_ref, wh_ref, bh_ref,
              wc_ref, bc_ref, h_ref, l_ref):
    deg = degT_ref[...]
    dis = lax.rsqrt(deg[:, 0:1] + deg[:, 1:2] + 1.0)
    agg = (acc_ref[0] + acc_ref[1] + y_ref[...]) * dis + bg_ref[...]
    x1 = jnp.maximum(agg, 0.0)
    cat = jnp.concatenate([x_ref[...], x1], axis=1)
    h = jnp.dot(cat, wh_ref[...], preferred_element_type=jnp.float32)
    h = jnp.maximum(h + bh_ref[...], 0.0)
    h_ref[...] = h
    l_ref[...] = jnp.dot(h, wc_ref[...],
                         preferred_element_type=jnp.float32) + bc_ref[...]


def _mlp_call(x, y, accp, degT, b_gcn, W_h, b_h, W_c, b_c):
    return pl.pallas_call(
        _mlp_body,
        grid=(N // BLK,),
        in_specs=[
            pl.BlockSpec((BLK, D), lambda i: (i, 0)),
            pl.BlockSpec((BLK, D), lambda i: (i, 0)),
            pl.BlockSpec((NC, BLK, D), lambda i: (0, i, 0)),
            pl.BlockSpec((BLK, 2), lambda i: (i, 0)),
            pl.BlockSpec((D,), lambda i: (0,)),
            pl.BlockSpec((2 * D, H), lambda i: (0, 0)),
            pl.BlockSpec((H,), lambda i: (0,)),
            pl.BlockSpec((H, C), lambda i: (0, 0)),
            pl.BlockSpec((C,), lambda i: (0,)),
        ],
        out_specs=[
            pl.BlockSpec((BLK, H), lambda i: (i, 0)),
            pl.BlockSpec((BLK, C), lambda i: (i, 0)),
        ],
        out_shape=[
            jax.ShapeDtypeStruct((N, H), jnp.float32),
            jax.ShapeDtypeStruct((N, C), jnp.float32),
        ],
    )(x, y, accp, degT, b_gcn, W_h, b_h, W_c, b_c)


def kernel(x, edge_index, img_sizes, W_gcn, b_gcn, W_h, b_h, W_c, b_c):
    ei = edge_index.astype(jnp.int32)
    ones_c = jnp.ones((CH,), jnp.float32)
    zeros_c = jnp.zeros((PERS,), jnp.float32)
    zrows_c = jnp.zeros((CH, D), jnp.float32)
    xw = _xw_call(x, W_gcn)
    degp = _deg_kernel(ei, ones_c, zeros_c)
    degT = jnp.swapaxes(degp, 0, 1)
    y = _scale_call(xw, degT)
    accp = _agg_kernel(y, ei, zrows_c)
    h, logits = _mlp_call(x, y, accp, degT,
                          b_gcn, W_h, b_h, W_c, b_c)
    return (h, logits)


# speedup vs baseline: 1.0361x; 1.0288x over previous
import functools

import jax
import jax.numpy as jnp
from jax import lax
from jax.experimental import pallas as pl
from jax.experimental.pallas import tpu as pltpu
from jax.experimental.pallas import tpu_sc as plsc

N = 10000
D = 128
E = 320000
NC, NS = 2, 16
NW = NC * NS
NP = 10240
CH = 128
NCHUNK = E // CH
NJ = NCHUNK // NW
NEXTRA = NCHUNK - NJ * NW
G = 6
NGRP = NJ // G
PERS = NP // NS

_mesh = plsc.VectorSubcoreMesh(core_axis_name="c", subcore_axis_name="s")


@functools.partial(
    pl.kernel,
    out_type=jax.ShapeDtypeStruct((NC, NP), jnp.float32),
    mesh=_mesh,
    scratch_types=[
        pltpu.VMEM((NJ + 1, CH), jnp.int32),
        pltpu.VMEM((CH,), jnp.float32),
        pltpu.VMEM((PERS,), jnp.float32),
        pltpu.VMEM_SHARED((NP,), jnp.float32),
        pltpu.SemaphoreType.DMA,
    ],
)
def _deg_kernel(ef_hbm, out_hbm, idx_v, ones_v, zb_v, deg_sh, sem):
    c = lax.axis_index("c")
    s = lax.axis_index("s")
    w = c * NS + s
    nj = jnp.where(w < NEXTRA, NJ + 1, NJ)

    def init_ones(i, _):
        ones_v[pl.ds(i * 16, 16)] = jnp.ones((16,), jnp.float32)
        return 0
    lax.fori_loop(0, CH // 16, init_ones, 0)

    def init_z(i, _):
        zb_v[pl.ds(i * 16, 16)] = jnp.zeros((16,), jnp.float32)
        return 0
    lax.fori_loop(0, PERS // 16, init_z, 0)

    pltpu.sync_copy(zb_v, deg_sh.at[pl.ds(s * PERS, PERS)])

    def load(r, _):
        pltpu.async_copy(ef_hbm.at[1, pl.ds(pl.multiple_of((w * NJ + r) * CH, CH), CH)],
                         idx_v.at[r], sem)
        return 0
    lax.fori_loop(0, NJ, load, 0)

    @pl.when(w < NEXTRA)
    def _():
        pltpu.async_copy(ef_hbm.at[1, pl.ds(pl.multiple_of((NW * NJ + w) * CH, CH), CH)],
                         idx_v.at[NJ], sem)

    def drain(r, _):
        pltpu.make_async_copy(ef_hbm.at[1, pl.ds(0, CH)], idx_v.at[0],
                              sem).wait()
        return 0
    lax.fori_loop(0, nj, drain, 0)
    plsc.subcore_barrier()

    def body(j, _):
        pltpu.sync_copy(ones_v, deg_sh.at[idx_v.at[j]], add=True)
        return 0
    lax.fori_loop(0, nj, body, 0)

    plsc.subcore_barrier()
    pltpu.sync_copy(deg_sh.at[pl.ds(s * PERS, PERS)],
                    out_hbm.at[c, pl.ds(s * PERS, PERS)])


@functools.partial(
    pl.kernel,
    out_type=jax.ShapeDtypeStruct((NC, NP, D), jnp.float32),
    mesh=_mesh,
    scratch_types=[
        pltpu.VMEM((2, G, CH), jnp.int32),
        pltpu.VMEM((2, G, CH), jnp.int32),
        pltpu.VMEM((2, CH, D), jnp.float32),
        pltpu.VMEM_SHARED((NP, D), jnp.float32),
        pltpu.SemaphoreType.DMA,
        pltpu.SemaphoreType.DMA,
        pltpu.SemaphoreType.DMA,
        pltpu.SemaphoreType.DMA,
        pltpu.SemaphoreType.DMA,
    ],
)
def _agg_kernel(y_hbm, ef_hbm, out_hbm,
                si_v, di_v, rows_v, acc_sh, semi, sem0, sem1, ssem0, ssem1):
    c = lax.axis_index("c")
    s = lax.axis_index("s")
    w = c * NS + s
    sems = (sem0, sem1)
    ssems = (ssem0, ssem1)

    def fetch_group(g, slot):
        for r in range(G):
            pltpu.async_copy(
                ef_hbm.at[0, pl.ds(pl.multiple_of((w * NJ + g * G + r) * CH, CH), CH)],
                si_v.at[slot, r], semi)
            pltpu.async_copy(
                ef_hbm.at[1, pl.ds(pl.multiple_of((w * NJ + g * G + r) * CH, CH), CH)],
                di_v.at[slot, r], semi)

    def wait_group():
        for _ in range(2 * G):
            pltpu.make_async_copy(ef_hbm.at[0, pl.ds(0, CH)], si_v.at[0, 0],
                                  semi).wait()

    def zrow(r, _):
        def zcol(k, _):
            rows_v[0, r, pl.ds(k * 16, 16)] = jnp.zeros((16,), jnp.float32)
            return 0
        lax.fori_loop(0, D // 16, zcol, 0)
        return 0
    lax.fori_loop(0, CH, zrow, 0)

    def zcopy(k, _):
        pltpu.sync_copy(rows_v.at[0], acc_sh.at[pl.ds(s * PERS + k * CH, CH)])
        return 0
    lax.fori_loop(0, PERS // CH, zcopy, 0)

    fetch_group(0, 0)
    wait_group()
    plsc.subcore_barrier()
    pltpu.async_copy(y_hbm.at[si_v.at[0, 0]], rows_v.at[0], sem0)

    def group(g, _):
        gb = lax.rem(g, 2)
        gn = lax.rem(g + 1, 2)

        @pl.when(g + 1 < NGRP)
        def _():
            fetch_group(g + 1, gn)

        for j in range(G):
            b = j % 2
            nb = (j + 1) % 2
            pltpu.make_async_copy(y_hbm.at[si_v.at[gb, j]], rows_v.at[b],
                                  sems[b]).wait()
            pltpu.async_copy(rows_v.at[b], acc_sh.at[di_v.at[gb, j]],
                             ssems[b], add=True)
            if j == 0:
                @pl.when(g > 0)
                def _():
                    pltpu.make_async_copy(rows_v.at[nb],
                                          acc_sh.at[di_v.at[0, 0]],
                                          ssems[nb]).wait()
            else:
                pltpu.make_async_copy(rows_v.at[nb],
                                      acc_sh.at[di_v.at[0, 0]],
                                      ssems[nb]).wait()
            if j + 1 < G:
                pltpu.async_copy(y_hbm.at[si_v.at[gb, j + 1]], rows_v.at[nb],
                                 sems[nb])
            else:
                @pl.when(g + 1 < NGRP)
                def _():
                    wait_group()
                    pltpu.async_copy(y_hbm.at[si_v.at[gn, 0]], rows_v.at[nb],
                                     sems[nb])
        return 0
    lax.fori_loop(0, NGRP, group, 0)

    pltpu.make_async_copy(rows_v.at[1], acc_sh.at[di_v.at[0, 0]],
                          ssems[1]).wait()

    @pl.when(w < NEXTRA)
    def _():
        pltpu.async_copy(ef_hbm.at[0, pl.ds(pl.multiple_of((NW * NJ + w) * CH, CH), CH)],
                         si_v.at[0, 0], semi)
        pltpu.async_copy(ef_hbm.at[1, pl.ds(pl.multiple_of((NW * NJ + w) * CH, CH), CH)],
                         di_v.at[0, 0], semi)
        pltpu.make_async_copy(ef_hbm.at[0, pl.ds(0, CH)], si_v.at[0, 0],
                              semi).wait()
        pltpu.make_async_copy(ef_hbm.at[0, pl.ds(0, CH)], si_v.at[0, 0],
                              semi).wait()
        pltpu.async_copy(y_hbm.at[si_v.at[0, 0]], rows_v.at[0], sem0).wait()
        pltpu.sync_copy(rows_v.at[0], acc_sh.at[di_v.at[0, 0]], add=True)

    plsc.subcore_barrier()
    pltpu.sync_copy(acc_sh.at[pl.ds(s * PERS, PERS)],
                    out_hbm.at[c, pl.ds(s * PERS, PERS)])


BLK = 1000
H = 512
C = 79


def _xw_body(x_ref, w_ref, o_ref):
    o_ref[...] = jnp.dot(x_ref[...], w_ref[...],
                         preferred_element_type=jnp.float32)


def _xw_call(x, W_gcn):
    return pl.pallas_call(
        _xw_body,
        grid=(N // BLK,),
        in_specs=[
            pl.BlockSpec((BLK, D), lambda i: (i, 0)),
            pl.BlockSpec((D, D), lambda i: (0, 0)),
        ],
        out_specs=pl.BlockSpec((BLK, D), lambda i: (i, 0)),
        out_shape=jax.ShapeDtypeStruct((N, D), jnp.float32),
    )(x, W_gcn)


def _scale_body(xw_ref, degT_ref, y_ref):
    deg = degT_ref[...]
    dis = lax.rsqrt(deg[:, 0:1] + deg[:, 1:2] + 1.0)
    y_ref[...] = xw_ref[...] * dis


def _scale_call(xw, degT):
    return pl.pallas_call(
        _scale_body,
        grid=(N // BLK,),
        in_specs=[
            pl.BlockSpec((BLK, D), lambda i: (i, 0)),
            pl.BlockSpec((BLK, 2), lambda i: (i, 0)),
        ],
        out_specs=pl.BlockSpec((BLK, D), lambda i: (i, 0)),
        out_shape=jax.ShapeDtypeStruct((N, D), jnp.float32),
    )(xw, degT)


def _mlp_body(x_ref, y_ref, acc_ref, degT_ref, bg_ref, wh_ref, bh_ref,
              wc_ref, bc_ref, h_ref, l_ref):
    deg = degT_ref[...]
    dis = lax.rsqrt(deg[:, 0:1] + deg[:, 1:2] + 1.0)
    agg = (acc_ref[0] + acc_ref[1] + y_ref[...]) * dis + bg_ref[...]
    x1 = jnp.maximum(agg, 0.0)
    cat = jnp.concatenate([x_ref[...], x1], axis=1)
    h = jnp.dot(cat, wh_ref[...], preferred_element_type=jnp.float32)
    h = jnp.maximum(h + bh_ref[...], 0.0)
    h_ref[...] = h
    l_ref[...] = jnp.dot(h, wc_ref[...],
                         preferred_element_type=jnp.float32) + bc_ref[...]


def _mlp_call(x, y, accp, degT, b_gcn, W_h, b_h, W_c, b_c):
    return pl.pallas_call(
        _mlp_body,
        grid=(N // BLK,),
        in_specs=[
            pl.BlockSpec((BLK, D), lambda i: (i, 0)),
            pl.BlockSpec((BLK, D), lambda i: (i, 0)),
            pl.BlockSpec((NC, BLK, D), lambda i: (0, i, 0)),
            pl.BlockSpec((BLK, 2), lambda i: (i, 0)),
            pl.BlockSpec((D,), lambda i: (0,)),
            pl.BlockSpec((2 * D, H), lambda i: (0, 0)),
            pl.BlockSpec((H,), lambda i: (0,)),
            pl.BlockSpec((H, C), lambda i: (0, 0)),
            pl.BlockSpec((C,), lambda i: (0,)),
        ],
        out_specs=[
            pl.BlockSpec((BLK, H), lambda i: (i, 0)),
            pl.BlockSpec((BLK, C), lambda i: (i, 0)),
        ],
        out_shape=[
            jax.ShapeDtypeStruct((N, H), jnp.float32),
            jax.ShapeDtypeStruct((N, C), jnp.float32),
        ],
    )(x, y, accp, degT, b_gcn, W_h, b_h, W_c, b_c)


def kernel(x, edge_index, img_sizes, W_gcn, b_gcn, W_h, b_h, W_c, b_c):
    ei = edge_index.astype(jnp.int32)
    xw = _xw_call(x, W_gcn)
    degp = _deg_kernel(ei)
    degT = jnp.swapaxes(degp, 0, 1)
    y = _scale_call(xw, degT)
    accp = _agg_kernel(y, ei)
    h, logits = _mlp_call(x, y, accp, degT,
                          b_gcn, W_h, b_h, W_c, b_c)
    return (h, logits)
